# Initial kernel scaffold; baseline (speedup 1.0000x reference)
import jax
import jax.numpy as jnp
from jax.experimental import pallas as pl


def kernel(node_feat, edge_feat, edge_index, params):
    raise NotImplementedError("write your pallas kernel here")


# speedup vs baseline: 23.8995x; 23.8995x over previous
import jax
import jax.numpy as jnp
from jax import lax
from jax.experimental import pallas as pl
from jax.experimental.pallas import tpu as pltpu
from jax.experimental.pallas import tpu_sc as plsc

NN = 50000
NE = 800000
ROW = 80
NEG = 0.01
_EPS = 1e-5

NC = 2
NS = 16
NW = NC * NS

HALF = NN // NC
DUMMY = HALF

G_CH = 200
G_SPAN = NE // NW
G_NCH = G_SPAN // G_CH

S_CH = 400
S_SPAN = NE // NS
S_NCH = S_SPAN // S_CH

R_CH = 200
R_NCH = HALF // R_CH


def _lrelu(v):
    return jnp.maximum(v, NEG * v)


def _kron5(w):
    return jnp.kron(jnp.eye(5, dtype=w.dtype), w)


def _rep16(v):
    return jnp.repeat(v, 16)


def _tile5(v):
    return jnp.tile(v, 5)


def _embed_body(x_ref, w_ref, b_ref, o_ref):
    o_ref[...] = jnp.dot(x_ref[...], w_ref[...],
                         preferred_element_type=jnp.float32) + b_ref[...]


def _tc_embed(nf, w, b, blk=1000):
    n = nf.shape[0]
    fo = w.shape[1]
    return pl.pallas_call(
        _embed_body,
        grid=(n // blk,),
        in_specs=[
            pl.BlockSpec((blk, nf.shape[1]), lambda i: (i, 0)),
            pl.BlockSpec(w.shape, lambda i: (0, 0)),
            pl.BlockSpec((1, fo), lambda i: (0, 0)),
        ],
        out_specs=pl.BlockSpec((blk, fo), lambda i: (i, 0)),
        out_shape=jax.ShapeDtypeStruct((n, fo), jnp.float32),
    )(nf, w, b.reshape(1, fo))


def _ab_body(x_ref, wa_ref, wb_ref, a_ref, b_ref):
    x = x_ref[...]
    a_ref[...] = jnp.dot(x, wa_ref[...], preferred_element_type=jnp.float32)
    b_ref[...] = jnp.dot(x, wb_ref[...], preferred_element_type=jnp.float32)


def _tc_ab(x, wa, wb, blk=1000):
    n = x.shape[0]
    return pl.pallas_call(
        _ab_body,
        grid=(n // blk,),
        in_specs=[
            pl.BlockSpec((blk, ROW), lambda i: (i, 0)),
            pl.BlockSpec((ROW, ROW), lambda i: (0, 0)),
            pl.BlockSpec((ROW, ROW), lambda i: (0, 0)),
        ],
        out_specs=[
            pl.BlockSpec((blk, ROW), lambda i: (i, 0)),
            pl.BlockSpec((blk, ROW), lambda i: (i, 0)),
        ],
        out_shape=[
            jax.ShapeDtypeStruct((n, ROW), jnp.float32),
            jax.ShapeDtypeStruct((n, ROW), jnp.float32),
        ],
    )(x, wa, wb)


def _edge_body(g_ref, ea_ref, cw_ref, cb_ref, w2_ref, b2_ref, o_ref):
    m1 = g_ref[...] + jnp.dot(ea_ref[...], cw_ref[...],
                              preferred_element_type=jnp.float32) + cb_ref[...]
    m1 = _lrelu(m1)
    m2 = jnp.dot(m1, w2_ref[...], preferred_element_type=jnp.float32) + b2_ref[...]
    o_ref[...] = _lrelu(m2)


def _tc_edge(g, ea, cw, cb, w2, b2, blk=1000):
    e = g.shape[0]
    return pl.pallas_call(
        _edge_body,
        grid=(e // blk,),
        in_specs=[
            pl.BlockSpec((blk, ROW), lambda i: (i, 0)),
            pl.BlockSpec((blk, 20), lambda i: (i, 0)),
            pl.BlockSpec((20, ROW), lambda i: (0, 0)),
            pl.BlockSpec((1, ROW), lambda i: (0, 0)),
            pl.BlockSpec((ROW, ROW), lambda i: (0, 0)),
            pl.BlockSpec((1, ROW), lambda i: (0, 0)),
        ],
        out_specs=pl.BlockSpec((blk, ROW), lambda i: (i, 0)),
        out_shape=jax.ShapeDtypeStruct((e, ROW), jnp.float32),
    )(g, ea, cw, cb.reshape(1, ROW), w2, b2.reshape(1, ROW))


def _update_body(x_ref, ag_ref, w1x_ref, w1a_ref, s1_ref, c1_ref,
                 w2_ref, s2_ref, c2_ref, o_ref):
    x = x_ref[...]
    u = (jnp.dot(x, w1x_ref[...], preferred_element_type=jnp.float32)
         + jnp.dot(ag_ref[...], w1a_ref[...], preferred_element_type=jnp.float32))
    u = _lrelu(u * s1_ref[...] + c1_ref[...])
    u = jnp.dot(u, w2_ref[...], preferred_element_type=jnp.float32)
    u = _lrelu(u * s2_ref[...] + c2_ref[...])
    o_ref[...] = x + u


def _tc_update(x, ag, w1x, w1a, s1, c1, w2, s2, c2, blk=1000):
    n = x.shape[0]
    wspec = pl.BlockSpec((ROW, ROW), lambda i: (0, 0))
    vspec = pl.BlockSpec((1, ROW), lambda i: (0, 0))
    return pl.pallas_call(
        _update_body,
        grid=(n // blk,),
        in_specs=[
            pl.BlockSpec((blk, ROW), lambda i: (i, 0)),
            pl.BlockSpec((blk, ROW), lambda i: (i, 0)),
            wspec, wspec, vspec, vspec, wspec, vspec, vspec,
        ],
        out_specs=pl.BlockSpec((blk, ROW), lambda i: (i, 0)),
        out_shape=jax.ShapeDtypeStruct((n, ROW), jnp.float32),
    )(x, ag, w1x, w1a, s1.reshape(1, ROW), c1.reshape(1, ROW),
      w2, s2.reshape(1, ROW), c2.reshape(1, ROW))


def _head_body(x_ref, pw_ref, ps_ref, pc_ref,
               w0_ref, s0_ref, c0_ref, w1_ref, s1_ref, c1_ref,
               w2_ref, s2_ref, c2_ref, w3_ref, s3_ref, c3_ref,
               fw_ref, o_ref):
    h = jnp.dot(x_ref[...], pw_ref[...], preferred_element_type=jnp.float32)
    h = _lrelu(h * ps_ref[...] + pc_ref[...])
    for w_ref, s_ref, c_ref in ((w0_ref, s0_ref, c0_ref),
                                (w1_ref, s1_ref, c1_ref),
                                (w2_ref, s2_ref, c2_ref),
                                (w3_ref, s3_ref, c3_ref)):
        h = jnp.dot(h, w_ref[...], preferred_element_type=jnp.float32)
        h = _lrelu(h * s_ref[...] + c_ref[...])
    h = jnp.dot(h, fw_ref[...], preferred_element_type=jnp.float32)
    o_ref[...] = jnp.sum(h[:, 0:16], axis=1, keepdims=True) * (1.0 / 16.0)


def _tc_head(x, pw, ps, pc, layers, fwp, blk=400):
    full = lambda a: pl.BlockSpec(a.shape, lambda i: (0, 0))
    n = x.shape[0]
    in_specs = [pl.BlockSpec((blk, ROW), lambda i: (i, 0)),
                full(pw), full(ps), full(pc)]
    args = [x, pw, ps, pc]
    for (w, s, c) in layers:
        in_specs += [full(w), full(s), full(c)]
        args += [w, s, c]
    in_specs.append(full(fwp))
    args.append(fwp)
    return pl.pallas_call(
        _head_body,
        grid=(n // blk,),
        in_specs=in_specs,
        out_specs=pl.BlockSpec((blk, 1), lambda i: (i, 0)),
        out_shape=jax.ShapeDtypeStruct((n, 1), jnp.float32),
    )(*args)


def _gather_body(a_hbm, b_hbm, dst_hbm, src_hbm, out_hbm,
                 idxd, idxs, bufa, bufb, sema, semb):
    w = lax.axis_index("s") * NC + lax.axis_index("c")
    base = w * G_SPAN

    def chunk(j, carry):
        off = base + j * G_CH
        pltpu.sync_copy(dst_hbm.at[pl.ds(off, G_CH)], idxd)
        pltpu.sync_copy(src_hbm.at[pl.ds(off, G_CH)], idxs)
        ca = pltpu.async_copy(a_hbm.at[idxd], bufa, sema)
        cb = pltpu.async_copy(b_hbm.at[idxs], bufb, semb)
        ca.wait()
        cb.wait()

        def add_row(e, c2):
            for k in range(ROW // 16):
                sl = pl.ds(k * 16, 16)
                bufa[e, sl] = bufa[e, sl] + bufb[e, sl]
            return c2

        lax.fori_loop(0, G_CH, add_row, 0, unroll=2)
        pltpu.sync_copy(bufa, out_hbm.at[pl.ds(off, G_CH)])
        return carry

    lax.fori_loop(0, G_NCH, chunk, 0)


def _sc_gather(a, b, dst, src):
    mesh = plsc.VectorSubcoreMesh(core_axis_name="c", subcore_axis_name="s")
    return pl.kernel(
        _gather_body,
        out_type=jax.ShapeDtypeStruct((NE, ROW), jnp.float32),
        mesh=mesh,
        scratch_types=[
            pltpu.VMEM((G_CH,), jnp.int32),
            pltpu.VMEM((G_CH,), jnp.int32),
            pltpu.VMEM((G_CH, ROW), jnp.float32),
            pltpu.VMEM((G_CH, ROW), jnp.float32),
            pltpu.SemaphoreType.DMA,
            pltpu.SemaphoreType.DMA,
        ],
        compiler_params=pltpu.CompilerParams(use_tc_tiling_on_sc=False),
    )(a, b, dst, src)


HROW = ROW // 2


def _scatter_body(m2_hbm, dst_hbm, z_hbm, out_hbm, acc_spm,
                  dbuf, libuf, mbuf, rbuf):
    c = lax.axis_index("c")
    s = lax.axis_index("s")
    node_base = c * HALF
    edge_base = s * S_SPAN

    pltpu.sync_copy(z_hbm, rbuf)

    for f in range(2):
        fo = f * HROW

        def zchunk(j, carry):
            k = s + NS * j

            @pl.when(k < R_NCH + 1)
            def _():
                pltpu.sync_copy(rbuf, acc_spm.at[pl.ds(k * R_CH, R_CH)])
            return carry

        lax.fori_loop(0, (R_NCH + 1 + NS - 1) // NS, zchunk, 0)
        plsc.subcore_barrier()

        def chunk(j, carry):
            off = edge_base + j * S_CH
            pltpu.sync_copy(dst_hbm.at[pl.ds(off, S_CH)], dbuf)
            pltpu.sync_copy(m2_hbm.at[pl.ds(off, S_CH), pl.ds(fo, HROW)], mbuf)

            def loc(i, c2):
                sl = pl.ds(i * 16, 16)
                v = dbuf[sl] - node_base
                ok = (v >= 0) & (v < HALF)
                libuf[sl] = jnp.where(ok, v, DUMMY)
                return c2

            lax.fori_loop(0, S_CH // 16, loc, 0, unroll=2)
            pltpu.sync_copy(mbuf, acc_spm.at[libuf], add=True)
            return carry

        lax.fori_loop(0, S_NCH, chunk, 0)
        plsc.subcore_barrier()

        def rchunk(j, carry):
            k = s + NS * j

            @pl.when(k < R_NCH)
            def _():
                pltpu.sync_copy(acc_spm.at[pl.ds(k * R_CH, R_CH)],
                                rbuf.at[pl.ds(0, R_CH)])
                pltpu.sync_copy(rbuf.at[pl.ds(0, R_CH)],
                                out_hbm.at[pl.ds(node_base + k * R_CH, R_CH),
                                           pl.ds(fo, HROW)])
            return carry

        lax.fori_loop(0, (R_NCH + NS - 1) // NS, rchunk, 0)
        plsc.subcore_barrier()
        pltpu.sync_copy(z_hbm, rbuf)


def _sc_scatter(m2, dst):
    mesh = plsc.VectorSubcoreMesh(core_axis_name="c", subcore_axis_name="s")
    zeros = jnp.zeros((R_CH, HROW), jnp.float32)
    return pl.kernel(
        _scatter_body,
        out_type=jax.ShapeDtypeStruct((NN, ROW), jnp.float32),
        mesh=mesh,
        scratch_types=[
            pltpu.VMEM_SHARED((HALF + 208, HROW), jnp.float32),
            pltpu.VMEM((S_CH,), jnp.int32),
            pltpu.VMEM((S_CH,), jnp.int32),
            pltpu.VMEM((S_CH, HROW), jnp.float32),
            pltpu.VMEM((R_CH, HROW), jnp.float32),
        ],
        compiler_params=pltpu.CompilerParams(use_tc_tiling_on_sc=False),
    )(m2, dst, zeros)


def kernel(node_feat, edge_feat, edge_index, params):
    nf = node_feat.reshape(NN, 35)
    ef = edge_feat.reshape(NE, 10)
    src = edge_index[0]
    dst = edge_index[1]

    inv = jnp.float32(1.0 / jnp.sqrt(1.0 + _EPS))

    x = _tc_embed(nf, _kron5(params["node_W"]), _tile5(params["node_b"]))
    ea = _tc_embed(ef, _kron5(params["edge_W"]), _tile5(params["edge_b"]))

    for cv in params["convs"]:
        wa = _kron5(cv["mW1"][0:16])
        wb = _kron5(cv["mW1"][16:32])
        cw = _kron5(cv["mW1"][32:36])
        cbias = _tile5(cv["mb1"])
        w2 = _kron5(cv["mW2"])
        b2 = _tile5(cv["mb2"])
        s1 = _rep16(cv["ug1"]) * inv
        c1 = _tile5(cv["ub1"]) * s1 + _rep16(cv["ubeta1"])
        s2 = _rep16(cv["ug2"]) * inv
        c2 = _tile5(cv["ub2"]) * s2 + _rep16(cv["ubeta2"])
        w1x = _kron5(cv["uW1"][0:16])
        w1a = _kron5(cv["uW1"][16:32])
        w2u = _kron5(cv["uW2"])

        a, b = _tc_ab(x, wa, wb)
        g = _sc_gather(a, b, dst, src)
        m2 = _tc_edge(g, ea, cw, cbias, w2, b2)
        aggr = _sc_scatter(m2, dst)
        x = _tc_update(x, aggr, w1x, w1a, s1, c1, w2u, s2, c2)

    ps = (params["pg"] * inv).reshape(1, -1)
    pc = (params["pb"] * (params["pg"] * inv) + params["pbeta"]).reshape(1, -1)
    layers = []
    for d in params["bind"]:
        s = d["g"] * inv
        layers.append((d["W"], s.reshape(1, -1), (d["b"] * s + d["beta"]).reshape(1, -1)))
    fwp = jnp.pad(params["fW"], ((0, 0), (0, 112)))
    fbias = jnp.mean(params["fb"])

    h = _tc_head(x, params["pW"], ps, pc, layers, fwp)
    return h + fbias


# speedup vs baseline: 27.2160x; 1.1388x over previous
import jax
import jax.numpy as jnp
from jax import lax
from jax.experimental import pallas as pl
from jax.experimental.pallas import tpu as pltpu
from jax.experimental.pallas import tpu_sc as plsc

NN = 50000
NE = 800000
ROW = 80
NEG = 0.01
_EPS = 1e-5

NC = 2
NS = 16
NW = NC * NS

HALF = NN // NC
DUMMY = HALF

G_CH = 200
G_SPAN = NE // NW
G_NCH = G_SPAN // G_CH

S_CH = 400
S_SPAN = NE // NS
S_NCH = S_SPAN // S_CH

R_CH = 200
R_NCH = HALF // R_CH


def _lrelu(v):
    return jnp.maximum(v, NEG * v)


def _kron5(w):
    return jnp.kron(jnp.eye(5, dtype=w.dtype), w)


def _rep16(v):
    return jnp.repeat(v, 16)


def _tile5(v):
    return jnp.tile(v, 5)


def _embed_body(x_ref, w_ref, b_ref, o_ref):
    o_ref[...] = jnp.dot(x_ref[...], w_ref[...],
                         preferred_element_type=jnp.float32) + b_ref[...]


def _tc_embed(nf, w, b, blk=5000):
    n = nf.shape[0]
    fo = w.shape[1]
    return pl.pallas_call(
        _embed_body,
        grid=(n // blk,),
        in_specs=[
            pl.BlockSpec((blk, nf.shape[1]), lambda i: (i, 0)),
            pl.BlockSpec(w.shape, lambda i: (0, 0)),
            pl.BlockSpec((1, fo), lambda i: (0, 0)),
        ],
        out_specs=pl.BlockSpec((blk, fo), lambda i: (i, 0)),
        out_shape=jax.ShapeDtypeStruct((n, fo), jnp.float32),
    )(nf, w, b.reshape(1, fo))


def _ab_body(x_ref, wa_ref, wb_ref, a_ref, b_ref):
    x = x_ref[...]
    a_ref[...] = jnp.dot(x, wa_ref[...], preferred_element_type=jnp.float32)
    b_ref[...] = jnp.dot(x, wb_ref[...], preferred_element_type=jnp.float32)


def _tc_ab(x, wa, wb, blk=5000):
    n = x.shape[0]
    return pl.pallas_call(
        _ab_body,
        grid=(n // blk,),
        in_specs=[
            pl.BlockSpec((blk, ROW), lambda i: (i, 0)),
            pl.BlockSpec((ROW, ROW), lambda i: (0, 0)),
            pl.BlockSpec((ROW, ROW), lambda i: (0, 0)),
        ],
        out_specs=[
            pl.BlockSpec((blk, ROW), lambda i: (i, 0)),
            pl.BlockSpec((blk, ROW), lambda i: (i, 0)),
        ],
        out_shape=[
            jax.ShapeDtypeStruct((n, ROW), jnp.float32),
            jax.ShapeDtypeStruct((n, ROW), jnp.float32),
        ],
    )(x, wa, wb)


def _edge_body(g_ref, ea_ref, cw_ref, cb_ref, w2_ref, b2_ref, o_ref):
    m1 = g_ref[...] + jnp.dot(ea_ref[...], cw_ref[...],
                              preferred_element_type=jnp.float32) + cb_ref[...]
    m1 = _lrelu(m1)
    m2 = jnp.dot(m1, w2_ref[...], preferred_element_type=jnp.float32) + b2_ref[...]
    o_ref[...] = _lrelu(m2)


def _tc_edge(g, ea, cw, cb, w2, b2, blk=8000):
    e = g.shape[0]
    return pl.pallas_call(
        _edge_body,
        grid=(e // blk,),
        in_specs=[
            pl.BlockSpec((blk, ROW), lambda i: (i, 0)),
            pl.BlockSpec((blk, 20), lambda i: (i, 0)),
            pl.BlockSpec((20, ROW), lambda i: (0, 0)),
            pl.BlockSpec((1, ROW), lambda i: (0, 0)),
            pl.BlockSpec((ROW, ROW), lambda i: (0, 0)),
            pl.BlockSpec((1, ROW), lambda i: (0, 0)),
        ],
        out_specs=pl.BlockSpec((blk, ROW), lambda i: (i, 0)),
        out_shape=jax.ShapeDtypeStruct((e, ROW), jnp.float32),
    )(g, ea, cw, cb.reshape(1, ROW), w2, b2.reshape(1, ROW))


def _update_body(x_ref, ag_ref, w1x_ref, w1a_ref, s1_ref, c1_ref,
                 w2_ref, s2_ref, c2_ref, o_ref):
    x = x_ref[...]
    u = (jnp.dot(x, w1x_ref[...], preferred_element_type=jnp.float32)
         + jnp.dot(ag_ref[...], w1a_ref[...], preferred_element_type=jnp.float32))
    u = _lrelu(u * s1_ref[...] + c1_ref[...])
    u = jnp.dot(u, w2_ref[...], preferred_element_type=jnp.float32)
    u = _lrelu(u * s2_ref[...] + c2_ref[...])
    o_ref[...] = x + u


def _tc_update(x, ag, w1x, w1a, s1, c1, w2, s2, c2, blk=5000):
    n = x.shape[0]
    wspec = pl.BlockSpec((ROW, ROW), lambda i: (0, 0))
    vspec = pl.BlockSpec((1, ROW), lambda i: (0, 0))
    return pl.pallas_call(
        _update_body,
        grid=(n // blk,),
        in_specs=[
            pl.BlockSpec((blk, ROW), lambda i: (i, 0)),
            pl.BlockSpec((blk, ROW), lambda i: (i, 0)),
            wspec, wspec, vspec, vspec, wspec, vspec, vspec,
        ],
        out_specs=pl.BlockSpec((blk, ROW), lambda i: (i, 0)),
        out_shape=jax.ShapeDtypeStruct((n, ROW), jnp.float32),
    )(x, ag, w1x, w1a, s1.reshape(1, ROW), c1.reshape(1, ROW),
      w2, s2.reshape(1, ROW), c2.reshape(1, ROW))


def _head_body(x_ref, pw_ref, ps_ref, pc_ref,
               w0_ref, s0_ref, c0_ref, w1_ref, s1_ref, c1_ref,
               w2_ref, s2_ref, c2_ref, w3_ref, s3_ref, c3_ref,
               fw_ref, o_ref):
    h = jnp.dot(x_ref[...], pw_ref[...], preferred_element_type=jnp.float32)
    h = _lrelu(h * ps_ref[...] + pc_ref[...])
    for w_ref, s_ref, c_ref in ((w0_ref, s0_ref, c0_ref),
                                (w1_ref, s1_ref, c1_ref),
                                (w2_ref, s2_ref, c2_ref),
                                (w3_ref, s3_ref, c3_ref)):
        h = jnp.dot(h, w_ref[...], preferred_element_type=jnp.float32)
        h = _lrelu(h * s_ref[...] + c_ref[...])
    h = jnp.dot(h, fw_ref[...], preferred_element_type=jnp.float32)
    o_ref[...] = jnp.sum(h[:, 0:16], axis=1, keepdims=True) * (1.0 / 16.0)


def _tc_head(x, pw, ps, pc, layers, fwp, blk=400):
    full = lambda a: pl.BlockSpec(a.shape, lambda i: (0, 0))
    n = x.shape[0]
    in_specs = [pl.BlockSpec((blk, ROW), lambda i: (i, 0)),
                full(pw), full(ps), full(pc)]
    args = [x, pw, ps, pc]
    for (w, s, c) in layers:
        in_specs += [full(w), full(s), full(c)]
        args += [w, s, c]
    in_specs.append(full(fwp))
    args.append(fwp)
    return pl.pallas_call(
        _head_body,
        grid=(n // blk,),
        in_specs=in_specs,
        out_specs=pl.BlockSpec((blk, 1), lambda i: (i, 0)),
        out_shape=jax.ShapeDtypeStruct((n, 1), jnp.float32),
    )(*args)


def _gather_body(a_hbm, b_hbm, dst_hbm, src_hbm, out_hbm,
                 idxd, idxs, bufa, bufb, sema, semb):
    w = lax.axis_index("s") * NC + lax.axis_index("c")
    base = w * G_SPAN

    def chunk(j, carry):
        off = base + j * G_CH
        pltpu.sync_copy(dst_hbm.at[pl.ds(off, G_CH)], idxd)
        pltpu.sync_copy(src_hbm.at[pl.ds(off, G_CH)], idxs)
        ca = pltpu.async_copy(a_hbm.at[idxd], bufa, sema)
        cb = pltpu.async_copy(b_hbm.at[idxs], bufb, semb)
        ca.wait()
        cb.wait()

        def add_row(e, c2):
            for k in range(ROW // 16):
                sl = pl.ds(k * 16, 16)
                bufa[e, sl] = bufa[e, sl] + bufb[e, sl]
            return c2

        lax.fori_loop(0, G_CH, add_row, 0, unroll=2)
        pltpu.sync_copy(bufa, out_hbm.at[pl.ds(off, G_CH)])
        return carry

    lax.fori_loop(0, G_NCH, chunk, 0)


def _sc_gather(a, b, dst, src):
    mesh = plsc.VectorSubcoreMesh(core_axis_name="c", subcore_axis_name="s")
    return pl.kernel(
        _gather_body,
        out_type=jax.ShapeDtypeStruct((NE, ROW), jnp.float32),
        mesh=mesh,
        scratch_types=[
            pltpu.VMEM((G_CH,), jnp.int32),
            pltpu.VMEM((G_CH,), jnp.int32),
            pltpu.VMEM((G_CH, ROW), jnp.float32),
            pltpu.VMEM((G_CH, ROW), jnp.float32),
            pltpu.SemaphoreType.DMA,
            pltpu.SemaphoreType.DMA,
        ],
        compiler_params=pltpu.CompilerParams(use_tc_tiling_on_sc=False),
    )(a, b, dst, src)


HROW = ROW // 2


def _scatter_body(m2_hbm, dst_hbm, z_hbm, out_hbm, acc_spm,
                  dbuf, libuf, mbuf, rbuf):
    c = lax.axis_index("c")
    s = lax.axis_index("s")
    node_base = c * HALF
    edge_base = s * S_SPAN

    pltpu.sync_copy(z_hbm, rbuf)

    for f in range(2):
        fo = f * HROW

        def zchunk(j, carry):
            k = s + NS * j

            @pl.when(k < R_NCH + 1)
            def _():
                pltpu.sync_copy(rbuf, acc_spm.at[pl.ds(k * R_CH, R_CH)])
            return carry

        lax.fori_loop(0, (R_NCH + 1 + NS - 1) // NS, zchunk, 0)
        plsc.subcore_barrier()

        def chunk(j, carry):
            off = edge_base + j * S_CH
            pltpu.sync_copy(dst_hbm.at[pl.ds(off, S_CH)], dbuf)
            pltpu.sync_copy(m2_hbm.at[pl.ds(off, S_CH), pl.ds(fo, HROW)], mbuf)

            def loc(i, c2):
                sl = pl.ds(i * 16, 16)
                v = dbuf[sl] - node_base
                ok = (v >= 0) & (v < HALF)
                libuf[sl] = jnp.where(ok, v, DUMMY)
                return c2

            lax.fori_loop(0, S_CH // 16, loc, 0, unroll=2)
            pltpu.sync_copy(mbuf, acc_spm.at[libuf], add=True)
            return carry

        lax.fori_loop(0, S_NCH, chunk, 0)
        plsc.subcore_barrier()

        def rchunk(j, carry):
            k = s + NS * j

            @pl.when(k < R_NCH)
            def _():
                pltpu.sync_copy(acc_spm.at[pl.ds(k * R_CH, R_CH)],
                                rbuf.at[pl.ds(0, R_CH)])
                pltpu.sync_copy(rbuf.at[pl.ds(0, R_CH)],
                                out_hbm.at[pl.ds(node_base + k * R_CH, R_CH),
                                           pl.ds(fo, HROW)])
            return carry

        lax.fori_loop(0, (R_NCH + NS - 1) // NS, rchunk, 0)
        plsc.subcore_barrier()
        pltpu.sync_copy(z_hbm, rbuf)


def _sc_scatter(m2, dst):
    mesh = plsc.VectorSubcoreMesh(core_axis_name="c", subcore_axis_name="s")
    zeros = jnp.zeros((R_CH, HROW), jnp.float32)
    return pl.kernel(
        _scatter_body,
        out_type=jax.ShapeDtypeStruct((NN, ROW), jnp.float32),
        mesh=mesh,
        scratch_types=[
            pltpu.VMEM_SHARED((HALF + 208, HROW), jnp.float32),
            pltpu.VMEM((S_CH,), jnp.int32),
            pltpu.VMEM((S_CH,), jnp.int32),
            pltpu.VMEM((S_CH, HROW), jnp.float32),
            pltpu.VMEM((R_CH, HROW), jnp.float32),
        ],
        compiler_params=pltpu.CompilerParams(use_tc_tiling_on_sc=False),
    )(m2, dst, zeros)


def kernel(node_feat, edge_feat, edge_index, params):
    nf = node_feat.reshape(NN, 35)
    ef = edge_feat.reshape(NE, 10)
    src = edge_index[0]
    dst = edge_index[1]

    inv = jnp.float32(1.0 / jnp.sqrt(1.0 + _EPS))

    x = _tc_embed(nf, _kron5(params["node_W"]), _tile5(params["node_b"]))
    ea = _tc_embed(ef, _kron5(params["edge_W"]), _tile5(params["edge_b"]))

    for cv in params["convs"]:
        wa = _kron5(cv["mW1"][0:16])
        wb = _kron5(cv["mW1"][16:32])
        cw = _kron5(cv["mW1"][32:36])
        cbias = _tile5(cv["mb1"])
        w2 = _kron5(cv["mW2"])
        b2 = _tile5(cv["mb2"])
        s1 = _rep16(cv["ug1"]) * inv
        c1 = _tile5(cv["ub1"]) * s1 + _rep16(cv["ubeta1"])
        s2 = _rep16(cv["ug2"]) * inv
        c2 = _tile5(cv["ub2"]) * s2 + _rep16(cv["ubeta2"])
        w1x = _kron5(cv["uW1"][0:16])
        w1a = _kron5(cv["uW1"][16:32])
        w2u = _kron5(cv["uW2"])

        a, b = _tc_ab(x, wa, wb)
        g = _sc_gather(a, b, dst, src)
        m2 = _tc_edge(g, ea, cw, cbias, w2, b2)
        aggr = _sc_scatter(m2, dst)
        x = _tc_update(x, aggr, w1x, w1a, s1, c1, w2u, s2, c2)

    ps = (params["pg"] * inv).reshape(1, -1)
    pc = (params["pb"] * (params["pg"] * inv) + params["pbeta"]).reshape(1, -1)
    layers = []
    for d in params["bind"]:
        s = d["g"] * inv
        layers.append((d["W"], s.reshape(1, -1), (d["b"] * s + d["beta"]).reshape(1, -1)))
    fwp = jnp.pad(params["fW"], ((0, 0), (0, 112)))
    fbias = jnp.mean(params["fb"])

    h = _tc_head(x, params["pW"], ps, pc, layers, fwp)
    return h + fbias


# speedup vs baseline: 30.3257x; 1.1143x over previous
import jax
import jax.numpy as jnp
from jax import lax
from jax.experimental import pallas as pl
from jax.experimental.pallas import tpu as pltpu
from jax.experimental.pallas import tpu_sc as plsc

NN = 50000
NE = 800000
ROW = 80
NEG = 0.01
_EPS = 1e-5

NC = 2
NS = 16
NW = NC * NS

HALF = NN // NC
DUMMY = HALF

G_CH = 200
G_SPAN = NE // NW
G_NCH = G_SPAN // G_CH

S_CH = 400
S_SPAN = NE // NS
S_NCH = S_SPAN // S_CH

R_CH = 200
R_NCH = HALF // R_CH


def _lrelu(v):
    return jnp.maximum(v, NEG * v)


def _kron5(w):
    return jnp.kron(jnp.eye(5, dtype=w.dtype), w)


def _rep16(v):
    return jnp.repeat(v, 16)


def _tile5(v):
    return jnp.tile(v, 5)


def _embed_body(x_ref, w_ref, b_ref, o_ref):
    o_ref[...] = jnp.dot(x_ref[...], w_ref[...],
                         preferred_element_type=jnp.float32) + b_ref[...]


def _tc_embed(nf, w, b, blk=5000):
    n = nf.shape[0]
    fo = w.shape[1]
    return pl.pallas_call(
        _embed_body,
        grid=(n // blk,),
        in_specs=[
            pl.BlockSpec((blk, nf.shape[1]), lambda i: (i, 0)),
            pl.BlockSpec(w.shape, lambda i: (0, 0)),
            pl.BlockSpec((1, fo), lambda i: (0, 0)),
        ],
        out_specs=pl.BlockSpec((blk, fo), lambda i: (i, 0)),
        out_shape=jax.ShapeDtypeStruct((n, fo), jnp.float32),
    )(nf, w, b.reshape(1, fo))


def _ab_body(x_ref, wa_ref, wb_ref, a_ref, b_ref):
    x = x_ref[...]
    a_ref[...] = jnp.dot(x, wa_ref[...], preferred_element_type=jnp.float32)
    b_ref[...] = jnp.dot(x, wb_ref[...], preferred_element_type=jnp.float32)


def _tc_ab(x, wa, wb, blk=5000):
    n = x.shape[0]
    return pl.pallas_call(
        _ab_body,
        grid=(n // blk,),
        in_specs=[
            pl.BlockSpec((blk, ROW), lambda i: (i, 0)),
            pl.BlockSpec((ROW, ROW), lambda i: (0, 0)),
            pl.BlockSpec((ROW, ROW), lambda i: (0, 0)),
        ],
        out_specs=[
            pl.BlockSpec((blk, ROW), lambda i: (i, 0)),
            pl.BlockSpec((blk, ROW), lambda i: (i, 0)),
        ],
        out_shape=[
            jax.ShapeDtypeStruct((n, ROW), jnp.float32),
            jax.ShapeDtypeStruct((n, ROW), jnp.float32),
        ],
    )(x, wa, wb)


def _edge_body(g_ref, ea_ref, cw_ref, cb_ref, w2_ref, b2_ref, o_ref):
    m1 = g_ref[...] + jnp.dot(ea_ref[...], cw_ref[...],
                              preferred_element_type=jnp.float32) + cb_ref[...]
    m1 = _lrelu(m1)
    m2 = jnp.dot(m1, w2_ref[...], preferred_element_type=jnp.float32) + b2_ref[...]
    o_ref[...] = _lrelu(m2)


def _tc_edge(g, ea, cw, cb, w2, b2, blk=8000):
    e = g.shape[0]
    return pl.pallas_call(
        _edge_body,
        grid=(e // blk,),
        in_specs=[
            pl.BlockSpec((blk, ROW), lambda i: (i, 0)),
            pl.BlockSpec((blk, 20), lambda i: (i, 0)),
            pl.BlockSpec((20, ROW), lambda i: (0, 0)),
            pl.BlockSpec((1, ROW), lambda i: (0, 0)),
            pl.BlockSpec((ROW, ROW), lambda i: (0, 0)),
            pl.BlockSpec((1, ROW), lambda i: (0, 0)),
        ],
        out_specs=pl.BlockSpec((blk, ROW), lambda i: (i, 0)),
        out_shape=jax.ShapeDtypeStruct((e, ROW), jnp.float32),
    )(g, ea, cw, cb.reshape(1, ROW), w2, b2.reshape(1, ROW))


def _update_body(x_ref, ag_ref, w1x_ref, w1a_ref, s1_ref, c1_ref,
                 w2_ref, s2_ref, c2_ref, o_ref):
    x = x_ref[...]
    u = (jnp.dot(x, w1x_ref[...], preferred_element_type=jnp.float32)
         + jnp.dot(ag_ref[...], w1a_ref[...], preferred_element_type=jnp.float32))
    u = _lrelu(u * s1_ref[...] + c1_ref[...])
    u = jnp.dot(u, w2_ref[...], preferred_element_type=jnp.float32)
    u = _lrelu(u * s2_ref[...] + c2_ref[...])
    o_ref[...] = x + u


def _tc_update(x, ag, w1x, w1a, s1, c1, w2, s2, c2, blk=5000):
    n = x.shape[0]
    wspec = pl.BlockSpec((ROW, ROW), lambda i: (0, 0))
    vspec = pl.BlockSpec((1, ROW), lambda i: (0, 0))
    return pl.pallas_call(
        _update_body,
        grid=(n // blk,),
        in_specs=[
            pl.BlockSpec((blk, ROW), lambda i: (i, 0)),
            pl.BlockSpec((blk, ROW), lambda i: (i, 0)),
            wspec, wspec, vspec, vspec, wspec, vspec, vspec,
        ],
        out_specs=pl.BlockSpec((blk, ROW), lambda i: (i, 0)),
        out_shape=jax.ShapeDtypeStruct((n, ROW), jnp.float32),
    )(x, ag, w1x, w1a, s1.reshape(1, ROW), c1.reshape(1, ROW),
      w2, s2.reshape(1, ROW), c2.reshape(1, ROW))


def _head_body(x_ref, pw_ref, ps_ref, pc_ref,
               w0_ref, s0_ref, c0_ref, w1_ref, s1_ref, c1_ref,
               w2_ref, s2_ref, c2_ref, w3_ref, s3_ref, c3_ref,
               fw_ref, o_ref):
    h = jnp.dot(x_ref[...], pw_ref[...], preferred_element_type=jnp.float32)
    h = _lrelu(h * ps_ref[...] + pc_ref[...])
    for w_ref, s_ref, c_ref in ((w0_ref, s0_ref, c0_ref),
                                (w1_ref, s1_ref, c1_ref),
                                (w2_ref, s2_ref, c2_ref),
                                (w3_ref, s3_ref, c3_ref)):
        h = jnp.dot(h, w_ref[...], preferred_element_type=jnp.float32)
        h = _lrelu(h * s_ref[...] + c_ref[...])
    h = jnp.dot(h, fw_ref[...], preferred_element_type=jnp.float32)
    o_ref[...] = jnp.sum(h[:, 0:16], axis=1, keepdims=True) * (1.0 / 16.0)


def _tc_head(x, pw, ps, pc, layers, fwp, blk=400):
    full = lambda a: pl.BlockSpec(a.shape, lambda i: (0, 0))
    n = x.shape[0]
    in_specs = [pl.BlockSpec((blk, ROW), lambda i: (i, 0)),
                full(pw), full(ps), full(pc)]
    args = [x, pw, ps, pc]
    for (w, s, c) in layers:
        in_specs += [full(w), full(s), full(c)]
        args += [w, s, c]
    in_specs.append(full(fwp))
    args.append(fwp)
    return pl.pallas_call(
        _head_body,
        grid=(n // blk,),
        in_specs=in_specs,
        out_specs=pl.BlockSpec((blk, 1), lambda i: (i, 0)),
        out_shape=jax.ShapeDtypeStruct((n, 1), jnp.float32),
    )(*args)


def _gather_body(a_hbm, b_hbm, dst_hbm, src_hbm, out_hbm,
                 idxd, idxs, bufa, bufb,
                 sa0, sa1, sb0, sb1, sd0, sd1, ss0, ss1):
    w = lax.axis_index("s") * NC + lax.axis_index("c")
    base = w * G_SPAN
    sa = (sa0, sa1)
    sb = (sb0, sb1)
    sd = (sd0, sd1)
    ss = (ss0, ss1)

    def issue_idx(cn, slot):
        off = base + cn * G_CH
        pltpu.async_copy(dst_hbm.at[pl.ds(off, G_CH)], idxd.at[slot], sd[slot])
        pltpu.async_copy(src_hbm.at[pl.ds(off, G_CH)], idxs.at[slot], ss[slot])

    def wait_idx(slot):
        pltpu.make_async_copy(dst_hbm.at[pl.ds(0, G_CH)], idxd.at[slot], sd[slot]).wait()
        pltpu.make_async_copy(src_hbm.at[pl.ds(0, G_CH)], idxs.at[slot], ss[slot]).wait()

    def issue_gather(slot):
        pltpu.async_copy(a_hbm.at[idxd.at[slot]], bufa.at[slot], sa[slot])
        pltpu.async_copy(b_hbm.at[idxs.at[slot]], bufb.at[slot], sb[slot])

    def wait_gather(slot):
        pltpu.make_async_copy(a_hbm.at[idxd.at[slot]], bufa.at[slot], sa[slot]).wait()
        pltpu.make_async_copy(b_hbm.at[idxs.at[slot]], bufb.at[slot], sb[slot]).wait()

    pltpu.sync_copy(dst_hbm.at[pl.ds(base, G_CH)], idxd.at[0])
    pltpu.sync_copy(src_hbm.at[pl.ds(base, G_CH)], idxs.at[0])
    issue_gather(0)
    issue_idx(1, 1)

    def outer(cc, carry):
        for p in range(2):
            c = cc * 2 + p
            q = 1 - p

            @pl.when(c < G_NCH)
            def _():
                @pl.when(c + 1 < G_NCH)
                def _():
                    wait_idx(q)
                    issue_gather(q)
                wait_gather(p)

                @pl.when(c + 2 < G_NCH)
                def _():
                    issue_idx(c + 2, p)

                ba = bufa.at[p]
                bb = bufb.at[p]

                def add_row(e, c2):
                    for k in range(ROW // 16):
                        sl = pl.ds(k * 16, 16)
                        ba[e, sl] = ba[e, sl] + bb[e, sl]
                    return c2

                lax.fori_loop(0, G_CH, add_row, 0, unroll=2)
                pltpu.sync_copy(ba, out_hbm.at[pl.ds(base + c * G_CH, G_CH)])
        return carry

    lax.fori_loop(0, (G_NCH + 1) // 2, outer, 0)


def _sc_gather(a, b, dst, src):
    mesh = plsc.VectorSubcoreMesh(core_axis_name="c", subcore_axis_name="s")
    return pl.kernel(
        _gather_body,
        out_type=jax.ShapeDtypeStruct((NE, ROW), jnp.float32),
        mesh=mesh,
        scratch_types=[
            pltpu.VMEM((2, G_CH), jnp.int32),
            pltpu.VMEM((2, G_CH), jnp.int32),
            pltpu.VMEM((2, G_CH, ROW), jnp.float32),
            pltpu.VMEM((2, G_CH, ROW), jnp.float32),
        ] + [pltpu.SemaphoreType.DMA] * 8,
        compiler_params=pltpu.CompilerParams(use_tc_tiling_on_sc=False),
    )(a, b, dst, src)


HROW = ROW // 2


def _scatter_body(m2_hbm, dst_hbm, z_hbm, out_hbm, acc_spm,
                  dbuf, libuf, mbuf, rbuf):
    c = lax.axis_index("c")
    s = lax.axis_index("s")
    node_base = c * HALF
    edge_base = s * S_SPAN

    pltpu.sync_copy(z_hbm, rbuf)

    for f in range(2):
        fo = f * HROW

        def zchunk(j, carry):
            k = s + NS * j

            @pl.when(k < R_NCH + 1)
            def _():
                pltpu.sync_copy(rbuf, acc_spm.at[pl.ds(k * R_CH, R_CH)])
            return carry

        lax.fori_loop(0, (R_NCH + 1 + NS - 1) // NS, zchunk, 0)
        plsc.subcore_barrier()

        def chunk(j, carry):
            off = edge_base + j * S_CH
            pltpu.sync_copy(dst_hbm.at[pl.ds(off, S_CH)], dbuf)
            pltpu.sync_copy(m2_hbm.at[pl.ds(off, S_CH), pl.ds(fo, HROW)], mbuf)

            def loc(i, c2):
                sl = pl.ds(i * 16, 16)
                v = dbuf[sl] - node_base
                ok = (v >= 0) & (v < HALF)
                libuf[sl] = jnp.where(ok, v, DUMMY)
                return c2

            lax.fori_loop(0, S_CH // 16, loc, 0, unroll=2)
            pltpu.sync_copy(mbuf, acc_spm.at[libuf], add=True)
            return carry

        lax.fori_loop(0, S_NCH, chunk, 0)
        plsc.subcore_barrier()

        def rchunk(j, carry):
            k = s + NS * j

            @pl.when(k < R_NCH)
            def _():
                pltpu.sync_copy(acc_spm.at[pl.ds(k * R_CH, R_CH)],
                                rbuf.at[pl.ds(0, R_CH)])
                pltpu.sync_copy(rbuf.at[pl.ds(0, R_CH)],
                                out_hbm.at[pl.ds(node_base + k * R_CH, R_CH),
                                           pl.ds(fo, HROW)])
            return carry

        lax.fori_loop(0, (R_NCH + NS - 1) // NS, rchunk, 0)
        plsc.subcore_barrier()
        pltpu.sync_copy(z_hbm, rbuf)


def _sc_scatter(m2, dst):
    mesh = plsc.VectorSubcoreMesh(core_axis_name="c", subcore_axis_name="s")
    zeros = jnp.zeros((R_CH, HROW), jnp.float32)
    return pl.kernel(
        _scatter_body,
        out_type=jax.ShapeDtypeStruct((NN, ROW), jnp.float32),
        mesh=mesh,
        scratch_types=[
            pltpu.VMEM_SHARED((HALF + 208, HROW), jnp.float32),
            pltpu.VMEM((S_CH,), jnp.int32),
            pltpu.VMEM((S_CH,), jnp.int32),
            pltpu.VMEM((S_CH, HROW), jnp.float32),
            pltpu.VMEM((R_CH, HROW), jnp.float32),
        ],
        compiler_params=pltpu.CompilerParams(use_tc_tiling_on_sc=False),
    )(m2, dst, zeros)


def kernel(node_feat, edge_feat, edge_index, params):
    nf = node_feat.reshape(NN, 35)
    ef = edge_feat.reshape(NE, 10)
    src = edge_index[0]
    dst = edge_index[1]

    inv = jnp.float32(1.0 / jnp.sqrt(1.0 + _EPS))

    x = _tc_embed(nf, _kron5(params["node_W"]), _tile5(params["node_b"]))
    ea = _tc_embed(ef, _kron5(params["edge_W"]), _tile5(params["edge_b"]))

    for cv in params["convs"]:
        wa = _kron5(cv["mW1"][0:16])
        wb = _kron5(cv["mW1"][16:32])
        cw = _kron5(cv["mW1"][32:36])
        cbias = _tile5(cv["mb1"])
        w2 = _kron5(cv["mW2"])
        b2 = _tile5(cv["mb2"])
        s1 = _rep16(cv["ug1"]) * inv
        c1 = _tile5(cv["ub1"]) * s1 + _rep16(cv["ubeta1"])
        s2 = _rep16(cv["ug2"]) * inv
        c2 = _tile5(cv["ub2"]) * s2 + _rep16(cv["ubeta2"])
        w1x = _kron5(cv["uW1"][0:16])
        w1a = _kron5(cv["uW1"][16:32])
        w2u = _kron5(cv["uW2"])

        a, b = _tc_ab(x, wa, wb)
        g = _sc_gather(a, b, dst, src)
        m2 = _tc_edge(g, ea, cw, cbias, w2, b2)
        aggr = _sc_scatter(m2, dst)
        x = _tc_update(x, aggr, w1x, w1a, s1, c1, w2u, s2, c2)

    ps = (params["pg"] * inv).reshape(1, -1)
    pc = (params["pb"] * (params["pg"] * inv) + params["pbeta"]).reshape(1, -1)
    layers = []
    for d in params["bind"]:
        s = d["g"] * inv
        layers.append((d["W"], s.reshape(1, -1), (d["b"] * s + d["beta"]).reshape(1, -1)))
    fwp = jnp.pad(params["fW"], ((0, 0), (0, 112)))
    fbias = jnp.mean(params["fb"])

    h = _tc_head(x, params["pW"], ps, pc, layers, fwp)
    return h + fbias


# speedup vs baseline: 30.3827x; 1.0019x over previous
import jax
import jax.numpy as jnp
from jax import lax
from jax.experimental import pallas as pl
from jax.experimental.pallas import tpu as pltpu
from jax.experimental.pallas import tpu_sc as plsc

NN = 50000
NE = 800000
ROW = 80
NEG = 0.01
_EPS = 1e-5

NC = 2
NS = 16
NW = NC * NS

HALF = NN // NC
DUMMY = HALF

G_CH = 200
G_SPAN = NE // NW
G_NCH = G_SPAN // G_CH

S_CH = 400
S_SPAN = NE // NS
S_NCH = S_SPAN // S_CH

R_CH = 200
R_NCH = HALF // R_CH


def _lrelu(v):
    return jnp.maximum(v, NEG * v)


def _kron5(w):
    return jnp.kron(jnp.eye(5, dtype=w.dtype), w)


def _rep16(v):
    return jnp.repeat(v, 16)


def _tile5(v):
    return jnp.tile(v, 5)


def _embed_body(x_ref, w_ref, b_ref, o_ref):
    o_ref[...] = jnp.dot(x_ref[...], w_ref[...],
                         preferred_element_type=jnp.float32) + b_ref[...]


def _tc_embed(nf, w, b, blk=5000):
    n = nf.shape[0]
    fo = w.shape[1]
    return pl.pallas_call(
        _embed_body,
        grid=(n // blk,),
        in_specs=[
            pl.BlockSpec((blk, nf.shape[1]), lambda i: (i, 0)),
            pl.BlockSpec(w.shape, lambda i: (0, 0)),
            pl.BlockSpec((1, fo), lambda i: (0, 0)),
        ],
        out_specs=pl.BlockSpec((blk, fo), lambda i: (i, 0)),
        out_shape=jax.ShapeDtypeStruct((n, fo), jnp.float32),
    )(nf, w, b.reshape(1, fo))


def _ab_body(x_ref, wa_ref, wb_ref, a_ref, b_ref):
    x = x_ref[...]
    a_ref[...] = jnp.dot(x, wa_ref[...], preferred_element_type=jnp.float32)
    b_ref[...] = jnp.dot(x, wb_ref[...], preferred_element_type=jnp.float32)


def _tc_ab(x, wa, wb, blk=5000):
    n = x.shape[0]
    return pl.pallas_call(
        _ab_body,
        grid=(n // blk,),
        in_specs=[
            pl.BlockSpec((blk, ROW), lambda i: (i, 0)),
            pl.BlockSpec((ROW, ROW), lambda i: (0, 0)),
            pl.BlockSpec((ROW, ROW), lambda i: (0, 0)),
        ],
        out_specs=[
            pl.BlockSpec((blk, ROW), lambda i: (i, 0)),
            pl.BlockSpec((blk, ROW), lambda i: (i, 0)),
        ],
        out_shape=[
            jax.ShapeDtypeStruct((n, ROW), jnp.float32),
            jax.ShapeDtypeStruct((n, ROW), jnp.float32),
        ],
    )(x, wa, wb)


def _edge_body(g_ref, ea_ref, cw_ref, cb_ref, w2_ref, b2_ref, o_ref):
    m1 = g_ref[...] + jnp.dot(ea_ref[...], cw_ref[...],
                              preferred_element_type=jnp.float32) + cb_ref[...]
    m1 = _lrelu(m1)
    m2 = jnp.dot(m1, w2_ref[...], preferred_element_type=jnp.float32) + b2_ref[...]
    o_ref[...] = _lrelu(m2)


def _tc_edge(g, ea, cw, cb, w2, b2, blk=8000):
    e = g.shape[0]
    return pl.pallas_call(
        _edge_body,
        grid=(e // blk,),
        in_specs=[
            pl.BlockSpec((blk, ROW), lambda i: (i, 0)),
            pl.BlockSpec((blk, 20), lambda i: (i, 0)),
            pl.BlockSpec((20, ROW), lambda i: (0, 0)),
            pl.BlockSpec((1, ROW), lambda i: (0, 0)),
            pl.BlockSpec((ROW, ROW), lambda i: (0, 0)),
            pl.BlockSpec((1, ROW), lambda i: (0, 0)),
        ],
        out_specs=pl.BlockSpec((blk, ROW), lambda i: (i, 0)),
        out_shape=jax.ShapeDtypeStruct((e, ROW), jnp.float32),
    )(g, ea, cw, cb.reshape(1, ROW), w2, b2.reshape(1, ROW))


def _update_body(x_ref, ag_ref, w1x_ref, w1a_ref, s1_ref, c1_ref,
                 w2_ref, s2_ref, c2_ref, o_ref):
    x = x_ref[...]
    u = (jnp.dot(x, w1x_ref[...], preferred_element_type=jnp.float32)
         + jnp.dot(ag_ref[...], w1a_ref[...], preferred_element_type=jnp.float32))
    u = _lrelu(u * s1_ref[...] + c1_ref[...])
    u = jnp.dot(u, w2_ref[...], preferred_element_type=jnp.float32)
    u = _lrelu(u * s2_ref[...] + c2_ref[...])
    o_ref[...] = x + u


def _tc_update(x, ag, w1x, w1a, s1, c1, w2, s2, c2, blk=5000):
    n = x.shape[0]
    wspec = pl.BlockSpec((ROW, ROW), lambda i: (0, 0))
    vspec = pl.BlockSpec((1, ROW), lambda i: (0, 0))
    return pl.pallas_call(
        _update_body,
        grid=(n // blk,),
        in_specs=[
            pl.BlockSpec((blk, ROW), lambda i: (i, 0)),
            pl.BlockSpec((blk, ROW), lambda i: (i, 0)),
            wspec, wspec, vspec, vspec, wspec, vspec, vspec,
        ],
        out_specs=pl.BlockSpec((blk, ROW), lambda i: (i, 0)),
        out_shape=jax.ShapeDtypeStruct((n, ROW), jnp.float32),
    )(x, ag, w1x, w1a, s1.reshape(1, ROW), c1.reshape(1, ROW),
      w2, s2.reshape(1, ROW), c2.reshape(1, ROW))


def _head_body(x_ref, pw_ref, ps_ref, pc_ref,
               w0_ref, s0_ref, c0_ref, w1_ref, s1_ref, c1_ref,
               w2_ref, s2_ref, c2_ref, w3_ref, s3_ref, c3_ref,
               fw_ref, o_ref):
    h = jnp.dot(x_ref[...], pw_ref[...], preferred_element_type=jnp.float32)
    h = _lrelu(h * ps_ref[...] + pc_ref[...])
    for w_ref, s_ref, c_ref in ((w0_ref, s0_ref, c0_ref),
                                (w1_ref, s1_ref, c1_ref),
                                (w2_ref, s2_ref, c2_ref),
                                (w3_ref, s3_ref, c3_ref)):
        h = jnp.dot(h, w_ref[...], preferred_element_type=jnp.float32)
        h = _lrelu(h * s_ref[...] + c_ref[...])
    h = jnp.dot(h, fw_ref[...], preferred_element_type=jnp.float32)
    o_ref[...] = jnp.sum(h[:, 0:16], axis=1, keepdims=True) * (1.0 / 16.0)


def _tc_head(x, pw, ps, pc, layers, fwp, blk=400):
    full = lambda a: pl.BlockSpec(a.shape, lambda i: (0, 0))
    n = x.shape[0]
    in_specs = [pl.BlockSpec((blk, ROW), lambda i: (i, 0)),
                full(pw), full(ps), full(pc)]
    args = [x, pw, ps, pc]
    for (w, s, c) in layers:
        in_specs += [full(w), full(s), full(c)]
        args += [w, s, c]
    in_specs.append(full(fwp))
    args.append(fwp)
    return pl.pallas_call(
        _head_body,
        grid=(n // blk,),
        in_specs=in_specs,
        out_specs=pl.BlockSpec((blk, 1), lambda i: (i, 0)),
        out_shape=jax.ShapeDtypeStruct((n, 1), jnp.float32),
    )(*args)


def _gather_body(a_hbm, b_hbm, dst_hbm, src_hbm, out_hbm,
                 idxd, idxs, bufa, bufb,
                 sa0, sa1, sb0, sb1, sd0, sd1, ss0, ss1):
    w = lax.axis_index("s") * NC + lax.axis_index("c")
    base = w * G_SPAN
    sa = (sa0, sa1)
    sb = (sb0, sb1)
    sd = (sd0, sd1)
    ss = (ss0, ss1)

    def issue_idx(cn, slot):
        off = base + cn * G_CH
        pltpu.async_copy(dst_hbm.at[pl.ds(off, G_CH)], idxd.at[slot], sd[slot])
        pltpu.async_copy(src_hbm.at[pl.ds(off, G_CH)], idxs.at[slot], ss[slot])

    def wait_idx(slot):
        pltpu.make_async_copy(dst_hbm.at[pl.ds(0, G_CH)], idxd.at[slot], sd[slot]).wait()
        pltpu.make_async_copy(src_hbm.at[pl.ds(0, G_CH)], idxs.at[slot], ss[slot]).wait()

    def issue_gather(slot):
        pltpu.async_copy(a_hbm.at[idxd.at[slot]], bufa.at[slot], sa[slot])
        pltpu.async_copy(b_hbm.at[idxs.at[slot]], bufb.at[slot], sb[slot])

    def wait_gather(slot):
        pltpu.make_async_copy(a_hbm.at[idxd.at[slot]], bufa.at[slot], sa[slot]).wait()
        pltpu.make_async_copy(b_hbm.at[idxs.at[slot]], bufb.at[slot], sb[slot]).wait()

    pltpu.sync_copy(dst_hbm.at[pl.ds(base, G_CH)], idxd.at[0])
    pltpu.sync_copy(src_hbm.at[pl.ds(base, G_CH)], idxs.at[0])
    issue_gather(0)
    issue_idx(1, 1)

    def outer(cc, carry):
        for p in range(2):
            c = cc * 2 + p
            q = 1 - p

            @pl.when(c < G_NCH)
            def _():
                @pl.when(c + 1 < G_NCH)
                def _():
                    wait_idx(q)
                    issue_gather(q)
                wait_gather(p)

                @pl.when(c + 2 < G_NCH)
                def _():
                    issue_idx(c + 2, p)

                ba = bufa.at[p]
                bb = bufb.at[p]

                def add_row(e, c2):
                    for k in range(ROW // 16):
                        sl = pl.ds(k * 16, 16)
                        ba[e, sl] = ba[e, sl] + bb[e, sl]
                    return c2

                lax.fori_loop(0, G_CH, add_row, 0, unroll=2)
                pltpu.sync_copy(ba, out_hbm.at[pl.ds(base + c * G_CH, G_CH)])
        return carry

    lax.fori_loop(0, (G_NCH + 1) // 2, outer, 0)


def _sc_gather(a, b, dst, src):
    mesh = plsc.VectorSubcoreMesh(core_axis_name="c", subcore_axis_name="s")
    return pl.kernel(
        _gather_body,
        out_type=jax.ShapeDtypeStruct((NE, ROW), jnp.float32),
        mesh=mesh,
        scratch_types=[
            pltpu.VMEM((2, G_CH), jnp.int32),
            pltpu.VMEM((2, G_CH), jnp.int32),
            pltpu.VMEM((2, G_CH, ROW), jnp.float32),
            pltpu.VMEM((2, G_CH, ROW), jnp.float32),
        ] + [pltpu.SemaphoreType.DMA] * 8,
        compiler_params=pltpu.CompilerParams(use_tc_tiling_on_sc=False),
    )(a, b, dst, src)


HROW = ROW // 2


def _scatter_body(m2_hbm, dst_hbm, z_hbm, out_hbm, acc_spm,
                  dbuf, libuf, mbuf, rbuf, sdt0, sdt1, smt0, smt1):
    c = lax.axis_index("c")
    s = lax.axis_index("s")
    node_base = c * HALF
    edge_base = s * S_SPAN
    sdt = (sdt0, sdt1)
    smt = (smt0, smt1)

    pltpu.sync_copy(z_hbm, rbuf)

    for f in range(2):
        fo = f * HROW

        def zchunk(j, carry):
            k = s + NS * j

            @pl.when(k < R_NCH + 1)
            def _():
                pltpu.sync_copy(rbuf, acc_spm.at[pl.ds(k * R_CH, R_CH)])
            return carry

        lax.fori_loop(0, (R_NCH + 1 + NS - 1) // NS, zchunk, 0)
        plsc.subcore_barrier()

        def issue_data(cn, slot):
            off = edge_base + cn * S_CH
            pltpu.async_copy(dst_hbm.at[pl.ds(off, S_CH)], dbuf.at[slot],
                             sdt[slot])
            pltpu.async_copy(m2_hbm.at[pl.ds(off, S_CH), pl.ds(fo, HROW)],
                             mbuf.at[slot], smt[slot])

        def wait_data(slot):
            pltpu.make_async_copy(dst_hbm.at[pl.ds(0, S_CH)], dbuf.at[slot],
                                  sdt[slot]).wait()
            pltpu.make_async_copy(m2_hbm.at[pl.ds(0, S_CH), pl.ds(fo, HROW)],
                                  mbuf.at[slot], smt[slot]).wait()

        issue_data(0, 0)

        def chunk(cc, carry):
            for p in range(2):
                j = cc * 2 + p
                q = 1 - p

                @pl.when(j < S_NCH)
                def _():
                    wait_data(p)

                    @pl.when(j + 1 < S_NCH)
                    def _():
                        issue_data(j + 1, q)

                    db = dbuf.at[p]
                    lb = libuf.at[p]

                    def loc(i, c2):
                        sl = pl.ds(i * 16, 16)
                        v = db[sl] - node_base
                        ok = (v >= 0) & (v < HALF)
                        lb[sl] = jnp.where(ok, v, DUMMY)
                        return c2

                    lax.fori_loop(0, S_CH // 16, loc, 0, unroll=2)
                    pltpu.sync_copy(mbuf.at[p], acc_spm.at[lb], add=True)
            return carry

        lax.fori_loop(0, (S_NCH + 1) // 2, chunk, 0)
        plsc.subcore_barrier()

        def rchunk(j, carry):
            k = s + NS * j

            @pl.when(k < R_NCH)
            def _():
                pltpu.sync_copy(acc_spm.at[pl.ds(k * R_CH, R_CH)],
                                rbuf.at[pl.ds(0, R_CH)])
                pltpu.sync_copy(rbuf.at[pl.ds(0, R_CH)],
                                out_hbm.at[pl.ds(node_base + k * R_CH, R_CH),
                                           pl.ds(fo, HROW)])
            return carry

        lax.fori_loop(0, (R_NCH + NS - 1) // NS, rchunk, 0)
        plsc.subcore_barrier()
        pltpu.sync_copy(z_hbm, rbuf)


def _sc_scatter(m2, dst):
    mesh = plsc.VectorSubcoreMesh(core_axis_name="c", subcore_axis_name="s")
    zeros = jnp.zeros((R_CH, HROW), jnp.float32)
    return pl.kernel(
        _scatter_body,
        out_type=jax.ShapeDtypeStruct((NN, ROW), jnp.float32),
        mesh=mesh,
        scratch_types=[
            pltpu.VMEM_SHARED((HALF + 208, HROW), jnp.float32),
            pltpu.VMEM((2, S_CH), jnp.int32),
            pltpu.VMEM((2, S_CH), jnp.int32),
            pltpu.VMEM((2, S_CH, HROW), jnp.float32),
            pltpu.VMEM((R_CH, HROW), jnp.float32),
        ] + [pltpu.SemaphoreType.DMA] * 4,
        compiler_params=pltpu.CompilerParams(use_tc_tiling_on_sc=False),
    )(m2, dst, zeros)


def kernel(node_feat, edge_feat, edge_index, params):
    nf = node_feat.reshape(NN, 35)
    ef = edge_feat.reshape(NE, 10)
    src = edge_index[0]
    dst = edge_index[1]

    inv = jnp.float32(1.0 / jnp.sqrt(1.0 + _EPS))

    x = _tc_embed(nf, _kron5(params["node_W"]), _tile5(params["node_b"]))
    ea = _tc_embed(ef, _kron5(params["edge_W"]), _tile5(params["edge_b"]))

    for cv in params["convs"]:
        wa = _kron5(cv["mW1"][0:16])
        wb = _kron5(cv["mW1"][16:32])
        cw = _kron5(cv["mW1"][32:36])
        cbias = _tile5(cv["mb1"])
        w2 = _kron5(cv["mW2"])
        b2 = _tile5(cv["mb2"])
        s1 = _rep16(cv["ug1"]) * inv
        c1 = _tile5(cv["ub1"]) * s1 + _rep16(cv["ubeta1"])
        s2 = _rep16(cv["ug2"]) * inv
        c2 = _tile5(cv["ub2"]) * s2 + _rep16(cv["ubeta2"])
        w1x = _kron5(cv["uW1"][0:16])
        w1a = _kron5(cv["uW1"][16:32])
        w2u = _kron5(cv["uW2"])

        a, b = _tc_ab(x, wa, wb)
        g = _sc_gather(a, b, dst, src)
        m2 = _tc_edge(g, ea, cw, cbias, w2, b2)
        aggr = _sc_scatter(m2, dst)
        x = _tc_update(x, aggr, w1x, w1a, s1, c1, w2u, s2, c2)

    ps = (params["pg"] * inv).reshape(1, -1)
    pc = (params["pb"] * (params["pg"] * inv) + params["pbeta"]).reshape(1, -1)
    layers = []
    for d in params["bind"]:
        s = d["g"] * inv
        layers.append((d["W"], s.reshape(1, -1), (d["b"] * s + d["beta"]).reshape(1, -1)))
    fwp = jnp.pad(params["fW"], ((0, 0), (0, 112)))
    fbias = jnp.mean(params["fb"])

    h = _tc_head(x, params["pW"], ps, pc, layers, fwp)
    return h + fbias


# speedup vs baseline: 42.6185x; 1.4027x over previous
import jax
import jax.numpy as jnp
from jax import lax
from jax.experimental import pallas as pl
from jax.experimental.pallas import tpu as pltpu
from jax.experimental.pallas import tpu_sc as plsc

NN = 50000
NE = 800000
ROW = 80
PAD = 128
NEG = 0.01
_EPS = 1e-5

NC = 2
NS = 16
NW = NC * NS

HALF = NN // NC
DUMMY = HALF

G_CH = 200
G_SPAN = NE // NW
G_NCH = G_SPAN // G_CH

S_CH = 400
S_SPAN = NE // NS
S_NCH = S_SPAN // S_CH
HROW = ROW // 2

R_CH = 200
R_NCH = HALF // R_CH


def _lrelu(v):
    return jnp.maximum(v, NEG * v)


def _kron5(w):
    return jnp.kron(jnp.eye(5, dtype=w.dtype), w)


def _rep16(v):
    return jnp.repeat(v, 16)


def _tile5(v):
    return jnp.tile(v, 5)


def _padc(w):
    return jnp.pad(w, ((0, 0), (0, PAD - w.shape[1])))


def _padrc(w):
    return jnp.pad(w, ((0, PAD - w.shape[0]), (0, PAD - w.shape[1])))


def _padv(v):
    return jnp.pad(v, (0, PAD - v.shape[0])).reshape(1, PAD)


def _embed_body(x_ref, w_ref, b_ref, o_ref):
    o_ref[...] = jnp.dot(x_ref[...], w_ref[...],
                         preferred_element_type=jnp.float32) + b_ref[...]


def _tc_embed(nf, w, b, blk=5000):
    n = nf.shape[0]
    fo = w.shape[1]
    return pl.pallas_call(
        _embed_body,
        grid=(n // blk,),
        in_specs=[
            pl.BlockSpec((blk, nf.shape[1]), lambda i: (i, 0)),
            pl.BlockSpec(w.shape, lambda i: (0, 0)),
            pl.BlockSpec((1, fo), lambda i: (0, 0)),
        ],
        out_specs=pl.BlockSpec((blk, fo), lambda i: (i, 0)),
        out_shape=jax.ShapeDtypeStruct((n, fo), jnp.float32),
    )(nf, w, b.reshape(1, fo))


def _embed_t_body(xt_ref, wt_ref, bt_ref, o_ref):
    o_ref[...] = jnp.dot(wt_ref[...], xt_ref[...],
                         preferred_element_type=jnp.float32) + bt_ref[...]


def _tc_embed_t(eft, wt, bt, blk=6400):
    e = eft.shape[1]
    return pl.pallas_call(
        _embed_t_body,
        grid=(e // blk,),
        in_specs=[
            pl.BlockSpec((10, blk), lambda i: (0, i)),
            pl.BlockSpec((20, 10), lambda i: (0, 0)),
            pl.BlockSpec((20, 1), lambda i: (0, 0)),
        ],
        out_specs=pl.BlockSpec((20, blk), lambda i: (0, i)),
        out_shape=jax.ShapeDtypeStruct((20, e), jnp.float32),
    )(eft, wt, bt.reshape(20, 1))


def _ab_body(x_ref, wa_ref, wb_ref, a_ref, b_ref):
    x = x_ref[...]
    a_ref[...] = jnp.dot(x, wa_ref[...], preferred_element_type=jnp.float32)
    b_ref[...] = jnp.dot(x, wb_ref[...], preferred_element_type=jnp.float32)


def _tc_ab(x, wa, wb, blk=5000):
    n = x.shape[0]
    return pl.pallas_call(
        _ab_body,
        grid=(n // blk,),
        in_specs=[
            pl.BlockSpec((blk, PAD), lambda i: (i, 0)),
            pl.BlockSpec((PAD, PAD), lambda i: (0, 0)),
            pl.BlockSpec((PAD, PAD), lambda i: (0, 0)),
        ],
        out_specs=[
            pl.BlockSpec((blk, PAD), lambda i: (i, 0)),
            pl.BlockSpec((blk, PAD), lambda i: (i, 0)),
        ],
        out_shape=[
            jax.ShapeDtypeStruct((n, PAD), jnp.float32),
            jax.ShapeDtypeStruct((n, PAD), jnp.float32),
        ],
    )(x, wa, wb)


def _edge_body(g_ref, ea_ref, cw_ref, cb_ref, w2_ref, b2_ref, o_ref):
    c = lax.dot_general(ea_ref[...], cw_ref[...], (((0,), (0,)), ((), ())),
                        preferred_element_type=jnp.float32)
    m1 = _lrelu(g_ref[...] + c + cb_ref[...])
    m2 = jnp.dot(m1, w2_ref[...], preferred_element_type=jnp.float32) + b2_ref[...]
    o_ref[...] = _lrelu(m2)


def _tc_edge(g, ea_t, cw, cb, w2, b2, blk=6400):
    e = g.shape[0]
    return pl.pallas_call(
        _edge_body,
        grid=(e // blk,),
        in_specs=[
            pl.BlockSpec((blk, PAD), lambda i: (i, 0)),
            pl.BlockSpec((20, blk), lambda i: (0, i)),
            pl.BlockSpec((20, PAD), lambda i: (0, 0)),
            pl.BlockSpec((1, PAD), lambda i: (0, 0)),
            pl.BlockSpec((PAD, PAD), lambda i: (0, 0)),
            pl.BlockSpec((1, PAD), lambda i: (0, 0)),
        ],
        out_specs=pl.BlockSpec((blk, PAD), lambda i: (i, 0)),
        out_shape=jax.ShapeDtypeStruct((e, PAD), jnp.float32),
    )(g, ea_t, cw, cb, w2, b2)


def _update_body(x_ref, ag_ref, w1x_ref, w1a_ref, s1_ref, c1_ref,
                 w2_ref, s2_ref, c2_ref, o_ref):
    x = x_ref[...]
    u = (jnp.dot(x, w1x_ref[...], preferred_element_type=jnp.float32)
         + jnp.dot(ag_ref[...], w1a_ref[...], preferred_element_type=jnp.float32))
    u = _lrelu(u * s1_ref[...] + c1_ref[...])
    u = jnp.dot(u, w2_ref[...], preferred_element_type=jnp.float32)
    u = _lrelu(u * s2_ref[...] + c2_ref[...])
    o_ref[...] = x + u


def _tc_update(x, ag, w1x, w1a, s1, c1, w2, s2, c2, blk=5000):
    n = x.shape[0]
    wspec = pl.BlockSpec((PAD, PAD), lambda i: (0, 0))
    vspec = pl.BlockSpec((1, PAD), lambda i: (0, 0))
    return pl.pallas_call(
        _update_body,
        grid=(n // blk,),
        in_specs=[
            pl.BlockSpec((blk, PAD), lambda i: (i, 0)),
            pl.BlockSpec((blk, PAD), lambda i: (i, 0)),
            wspec, wspec, vspec, vspec, wspec, vspec, vspec,
        ],
        out_specs=pl.BlockSpec((blk, PAD), lambda i: (i, 0)),
        out_shape=jax.ShapeDtypeStruct((n, PAD), jnp.float32),
    )(x, ag, w1x, w1a, s1, c1, w2, s2, c2)


def _head_body(x_ref, pw_ref, ps_ref, pc_ref,
               w0_ref, s0_ref, c0_ref, w1_ref, s1_ref, c1_ref,
               w2_ref, s2_ref, c2_ref, w3_ref, s3_ref, c3_ref,
               fw_ref, o_ref):
    h = jnp.dot(x_ref[...], pw_ref[...], preferred_element_type=jnp.float32)
    h = _lrelu(h * ps_ref[...] + pc_ref[...])
    for w_ref, s_ref, c_ref in ((w0_ref, s0_ref, c0_ref),
                                (w1_ref, s1_ref, c1_ref),
                                (w2_ref, s2_ref, c2_ref),
                                (w3_ref, s3_ref, c3_ref)):
        h = jnp.dot(h, w_ref[...], preferred_element_type=jnp.float32)
        h = _lrelu(h * s_ref[...] + c_ref[...])
    h = jnp.dot(h, fw_ref[...], preferred_element_type=jnp.float32)
    o_ref[...] = jnp.sum(h[:, 0:16], axis=1, keepdims=True) * (1.0 / 16.0)


def _tc_head(x, pw, ps, pc, layers, fwp, blk=400):
    full = lambda a: pl.BlockSpec(a.shape, lambda i: (0, 0))
    n = x.shape[0]
    in_specs = [pl.BlockSpec((blk, PAD), lambda i: (i, 0)),
                full(pw), full(ps), full(pc)]
    args = [x, pw, ps, pc]
    for (w, s, c) in layers:
        in_specs += [full(w), full(s), full(c)]
        args += [w, s, c]
    in_specs.append(full(fwp))
    args.append(fwp)
    return pl.pallas_call(
        _head_body,
        grid=(n // blk,),
        in_specs=in_specs,
        out_specs=pl.BlockSpec((blk, 1), lambda i: (i, 0)),
        out_shape=jax.ShapeDtypeStruct((n, 1), jnp.float32),
    )(*args)


def _gather_body(a_hbm, b_hbm, dst_hbm, src_hbm, out_hbm,
                 idxd, idxs, bufa, bufb,
                 sa0, sa1, sb0, sb1, sd0, sd1, ss0, ss1):
    w = lax.axis_index("s") * NC + lax.axis_index("c")
    base = w * G_SPAN
    sa = (sa0, sa1)
    sb = (sb0, sb1)
    sd = (sd0, sd1)
    ss = (ss0, ss1)

    def issue_idx(cn, slot):
        off = base + cn * G_CH
        pltpu.async_copy(dst_hbm.at[pl.ds(off, G_CH)], idxd.at[slot], sd[slot])
        pltpu.async_copy(src_hbm.at[pl.ds(off, G_CH)], idxs.at[slot], ss[slot])

    def wait_idx(slot):
        pltpu.make_async_copy(dst_hbm.at[pl.ds(0, G_CH)], idxd.at[slot], sd[slot]).wait()
        pltpu.make_async_copy(src_hbm.at[pl.ds(0, G_CH)], idxs.at[slot], ss[slot]).wait()

    def issue_gather(slot):
        pltpu.async_copy(a_hbm.at[idxd.at[slot]], bufa.at[slot], sa[slot])
        pltpu.async_copy(b_hbm.at[idxs.at[slot]], bufb.at[slot], sb[slot])

    def wait_gather(slot):
        pltpu.make_async_copy(a_hbm.at[idxd.at[slot]], bufa.at[slot], sa[slot]).wait()
        pltpu.make_async_copy(b_hbm.at[idxs.at[slot]], bufb.at[slot], sb[slot]).wait()

    pltpu.sync_copy(dst_hbm.at[pl.ds(base, G_CH)], idxd.at[0])
    pltpu.sync_copy(src_hbm.at[pl.ds(base, G_CH)], idxs.at[0])
    issue_gather(0)
    issue_idx(1, 1)

    def outer(cc, carry):
        for p in range(2):
            c = cc * 2 + p
            q = 1 - p

            @pl.when(c < G_NCH)
            def _():
                @pl.when(c + 1 < G_NCH)
                def _():
                    wait_idx(q)
                    issue_gather(q)
                wait_gather(p)

                @pl.when(c + 2 < G_NCH)
                def _():
                    issue_idx(c + 2, p)

                ba = bufa.at[p]
                bb = bufb.at[p]

                def add_row(e, c2):
                    for k in range(PAD // 16):
                        sl = pl.ds(k * 16, 16)
                        ba[e, sl] = ba[e, sl] + bb[e, sl]
                    return c2

                lax.fori_loop(0, G_CH, add_row, 0, unroll=2)
                pltpu.sync_copy(ba, out_hbm.at[pl.ds(base + c * G_CH, G_CH)])
        return carry

    lax.fori_loop(0, (G_NCH + 1) // 2, outer, 0)


def _sc_gather(a, b, dst, src):
    mesh = plsc.VectorSubcoreMesh(core_axis_name="c", subcore_axis_name="s")
    return pl.kernel(
        _gather_body,
        out_type=jax.ShapeDtypeStruct((NE, PAD), jnp.float32),
        mesh=mesh,
        scratch_types=[
            pltpu.VMEM((2, G_CH), jnp.int32),
            pltpu.VMEM((2, G_CH), jnp.int32),
            pltpu.VMEM((2, G_CH, PAD), jnp.float32),
            pltpu.VMEM((2, G_CH, PAD), jnp.float32),
        ] + [pltpu.SemaphoreType.DMA] * 8,
        compiler_params=pltpu.CompilerParams(use_tc_tiling_on_sc=False),
    )(a, b, dst, src)


def _scatter_body(m2_hbm, dst_hbm, z_hbm, out_hbm, acc_spm,
                  dbuf, libuf, mbuf, rbuf, sdt0, sdt1, smt0, smt1):
    c = lax.axis_index("c")
    s = lax.axis_index("s")
    node_base = c * HALF
    edge_base = s * S_SPAN
    sdt = (sdt0, sdt1)
    smt = (smt0, smt1)

    pltpu.sync_copy(z_hbm, rbuf)

    for f in range(2):
        fo = f * HROW

        def zchunk(j, carry):
            k = s + NS * j

            @pl.when(k < R_NCH + 1)
            def _():
                pltpu.sync_copy(rbuf.at[:, pl.ds(0, HROW)],
                                acc_spm.at[pl.ds(k * R_CH, R_CH)])
            return carry

        lax.fori_loop(0, (R_NCH + 1 + NS - 1) // NS, zchunk, 0)
        plsc.subcore_barrier()

        def issue_data(cn, slot):
            off = edge_base + cn * S_CH
            pltpu.async_copy(dst_hbm.at[pl.ds(off, S_CH)], dbuf.at[slot],
                             sdt[slot])
            pltpu.async_copy(m2_hbm.at[pl.ds(off, S_CH), pl.ds(fo, HROW)],
                             mbuf.at[slot], smt[slot])

        def wait_data(slot):
            pltpu.make_async_copy(dst_hbm.at[pl.ds(0, S_CH)], dbuf.at[slot],
                                  sdt[slot]).wait()
            pltpu.make_async_copy(m2_hbm.at[pl.ds(0, S_CH), pl.ds(fo, HROW)],
                                  mbuf.at[slot], smt[slot]).wait()

        issue_data(0, 0)

        def chunk(cc, carry):
            for p in range(2):
                j = cc * 2 + p
                q = 1 - p

                @pl.when(j < S_NCH)
                def _():
                    wait_data(p)

                    @pl.when(j + 1 < S_NCH)
                    def _():
                        issue_data(j + 1, q)

                    db = dbuf.at[p]
                    lb = libuf.at[p]

                    def loc(i, c2):
                        sl = pl.ds(i * 16, 16)
                        v = db[sl] - node_base
                        ok = (v >= 0) & (v < HALF)
                        lb[sl] = jnp.where(ok, v, DUMMY)
                        return c2

                    lax.fori_loop(0, S_CH // 16, loc, 0, unroll=2)
                    pltpu.sync_copy(mbuf.at[p], acc_spm.at[lb], add=True)
            return carry

        lax.fori_loop(0, (S_NCH + 1) // 2, chunk, 0)
        plsc.subcore_barrier()

        def rchunk(j, carry):
            k = s + NS * j

            @pl.when(k < R_NCH)
            def _():
                rows = pl.ds(node_base + k * R_CH, R_CH)
                pltpu.sync_copy(acc_spm.at[pl.ds(k * R_CH, R_CH)],
                                rbuf.at[:, pl.ds(0, HROW)])
                pltpu.sync_copy(rbuf.at[:, pl.ds(0, HROW)],
                                out_hbm.at[rows, pl.ds(fo, HROW)])
                if f == 1:
                    pltpu.sync_copy(rbuf.at[:, pl.ds(ROW, PAD - ROW)],
                                    out_hbm.at[rows, pl.ds(ROW, PAD - ROW)])
            return carry

        lax.fori_loop(0, (R_NCH + NS - 1) // NS, rchunk, 0)
        plsc.subcore_barrier()
        pltpu.sync_copy(z_hbm.at[:, pl.ds(0, HROW)], rbuf.at[:, pl.ds(0, HROW)])


def _sc_scatter(m2, dst):
    mesh = plsc.VectorSubcoreMesh(core_axis_name="c", subcore_axis_name="s")
    zeros = jnp.zeros((R_CH, PAD), jnp.float32)
    return pl.kernel(
        _scatter_body,
        out_type=jax.ShapeDtypeStruct((NN, PAD), jnp.float32),
        mesh=mesh,
        scratch_types=[
            pltpu.VMEM_SHARED((HALF + 208, HROW), jnp.float32),
            pltpu.VMEM((2, S_CH), jnp.int32),
            pltpu.VMEM((2, S_CH), jnp.int32),
            pltpu.VMEM((2, S_CH, HROW), jnp.float32),
            pltpu.VMEM((R_CH, PAD), jnp.float32),
        ] + [pltpu.SemaphoreType.DMA] * 4,
        compiler_params=pltpu.CompilerParams(use_tc_tiling_on_sc=False),
    )(m2, dst, zeros)


def kernel(node_feat, edge_feat, edge_index, params):
    nf = node_feat.reshape(NN, 35)
    eft = edge_feat.reshape(NE, 10).T
    src = edge_index[0]
    dst = edge_index[1]

    inv = jnp.float32(1.0 / jnp.sqrt(1.0 + _EPS))

    x = _tc_embed(nf, _padc(_kron5(params["node_W"])),
                  jnp.pad(_tile5(params["node_b"]), (0, PAD - ROW)))
    ea_t = _tc_embed_t(eft, _kron5(params["edge_W"]).T,
                       _tile5(params["edge_b"]))

    for cv in params["convs"]:
        wa = _padrc(_kron5(cv["mW1"][0:16]))
        wb = _padrc(_kron5(cv["mW1"][16:32]))
        cw = _padc(_kron5(cv["mW1"][32:36]))
        cbias = _padv(_tile5(cv["mb1"]))
        w2 = _padrc(_kron5(cv["mW2"]))
        b2 = _padv(_tile5(cv["mb2"]))
        s1 = _padv(_rep16(cv["ug1"]) * inv)
        c1 = _padv(_tile5(cv["ub1"]) * (_rep16(cv["ug1"]) * inv)
                   + _rep16(cv["ubeta1"]))
        s2 = _padv(_rep16(cv["ug2"]) * inv)
        c2 = _padv(_tile5(cv["ub2"]) * (_rep16(cv["ug2"]) * inv)
                   + _rep16(cv["ubeta2"]))
        w1x = _padrc(_kron5(cv["uW1"][0:16]))
        w1a = _padrc(_kron5(cv["uW1"][16:32]))
        w2u = _padrc(_kron5(cv["uW2"]))

        a, b = _tc_ab(x, wa, wb)
        g = _sc_gather(a, b, dst, src)
        m2 = _tc_edge(g, ea_t, cw, cbias, w2, b2)
        aggr = _sc_scatter(m2, dst)
        x = _tc_update(x, aggr, w1x, w1a, s1, c1, w2u, s2, c2)

    pw = jnp.pad(params["pW"], ((0, PAD - ROW), (0, 0)))
    ps = (params["pg"] * inv).reshape(1, -1)
    pc = (params["pb"] * (params["pg"] * inv) + params["pbeta"]).reshape(1, -1)
    layers = []
    for d in params["bind"]:
        s = d["g"] * inv
        layers.append((d["W"], s.reshape(1, -1),
                       (d["b"] * s + d["beta"]).reshape(1, -1)))
    fwp = jnp.pad(params["fW"], ((0, 0), (0, 112)))
    fbias = jnp.mean(params["fb"])

    h = _tc_head(x, pw, ps, pc, layers, fwp)
    return h + fbias
